# baseline (speedup 1.0000x reference)
import functools

import jax
import jax.numpy as jnp
from jax import lax
from jax.experimental import pallas as pl
from jax.experimental.pallas import tpu as pltpu
from jax.experimental.pallas import tpu_sc as plsc

BATCH = 16384
DIM = 32
RANK = 16

_NC, _NS = 2, 16
_NW = _NC * _NS
_BPW = BATCH // _NW
_CHUNK = 128
_NCHUNK = _BPW // _CHUNK


def _sc_gather(users_r, items_r, gu_t, nu_t, gi_t, ni_t, ga_t, na_t):
    mesh = plsc.VectorSubcoreMesh(core_axis_name="c", subcore_axis_name="s")
    f32 = jnp.float32

    @functools.partial(
        pl.kernel,
        mesh=mesh,
        compiler_params=pltpu.CompilerParams(use_tc_tiling_on_sc=False),
        out_type=[
            jax.ShapeDtypeStruct((BATCH, DIM), f32),
            jax.ShapeDtypeStruct((BATCH, DIM), f32),
            jax.ShapeDtypeStruct((BATCH, DIM), f32),
            jax.ShapeDtypeStruct((BATCH, DIM), f32),
            jax.ShapeDtypeStruct((BATCH, RANK), f32),
            jax.ShapeDtypeStruct((BATCH, RANK), f32),
        ],
        scratch_types=[
            pltpu.VMEM((_NCHUNK, _CHUNK), jnp.int32),
            pltpu.VMEM((_NCHUNK, _CHUNK), jnp.int32),
            pltpu.VMEM((_NCHUNK, _CHUNK, DIM), f32),
            pltpu.VMEM((_NCHUNK, _CHUNK, DIM), f32),
            pltpu.VMEM((_NCHUNK, _CHUNK, DIM), f32),
            pltpu.VMEM((_NCHUNK, _CHUNK, DIM), f32),
            pltpu.VMEM((_NCHUNK, _CHUNK, RANK), f32),
            pltpu.VMEM((_NCHUNK, _CHUNK, RANK), f32),
            pltpu.SemaphoreType.DMA,
        ],
    )
    def k(users_hbm, items_hbm, gu_hbm, nu_hbm, gi_hbm, ni_hbm, ga_hbm, na_hbm,
          o_gu, o_nu, o_gi, o_ni, o_ga, o_na,
          idx_u, idx_i, r_gu, r_nu, r_gi, r_ni, r_ga, r_na, sem):
        wid = lax.axis_index("s") * _NC + lax.axis_index("c")
        base = wid * _BPW
        pltpu.sync_copy(users_hbm.at[wid], idx_u)
        pltpu.sync_copy(items_hbm.at[wid], idx_i)
        copies = []
        for j in range(_NCHUNK):
            copies.append(pltpu.async_copy(gu_hbm.at[idx_u.at[j]], r_gu.at[j], sem))
            copies.append(pltpu.async_copy(nu_hbm.at[idx_u.at[j]], r_nu.at[j], sem))
            copies.append(pltpu.async_copy(gi_hbm.at[idx_i.at[j]], r_gi.at[j], sem))
            copies.append(pltpu.async_copy(ni_hbm.at[idx_i.at[j]], r_ni.at[j], sem))
            copies.append(pltpu.async_copy(ga_hbm.at[idx_i.at[j]], r_ga.at[j], sem))
            copies.append(pltpu.async_copy(na_hbm.at[idx_i.at[j]], r_na.at[j], sem))
        for c in copies:
            c.wait()
        for j in range(_NCHUNK):
            dst = pl.ds(base + j * _CHUNK, _CHUNK)
            pltpu.sync_copy(r_gu.at[j], o_gu.at[dst])
            pltpu.sync_copy(r_nu.at[j], o_nu.at[dst])
            pltpu.sync_copy(r_gi.at[j], o_gi.at[dst])
            pltpu.sync_copy(r_ni.at[j], o_ni.at[dst])
            pltpu.sync_copy(r_ga.at[j], o_ga.at[dst])
            pltpu.sync_copy(r_na.at[j], o_na.at[dst])

    return k(users_r, items_r, gu_t, nu_t, gi_t, ni_t, ga_t, na_t)


_BLK = 2048


def _dense_body(gu_ref, nu_ref, gi_ref, ni_ref, ga_ref, na_ref,
                gB_ref, nB_ref, w1t_ref, b1_ref, w2t_ref, b2_ref,
                wo_g_ref, wo_n_ref, out_ref):
    hi = jax.lax.Precision.HIGHEST
    dot = functools.partial(jnp.dot, precision=hi, preferred_element_type=jnp.float32)
    gi_adj = gi_ref[...] + dot(ga_ref[...], gB_ref[...])
    ni_adj = ni_ref[...] + dot(na_ref[...], nB_ref[...])
    h1 = dot(nu_ref[...], w1t_ref[0:DIM, :]) + dot(ni_adj, w1t_ref[DIM:2 * DIM, :])
    h1 = jnp.maximum(h1 + b1_ref[...], 0.0)
    h2 = jnp.maximum(dot(h1, w2t_ref[...]) + b2_ref[...], 0.0)
    gmf_h = gu_ref[...] * gi_adj
    pred = (jnp.sum(gmf_h * wo_g_ref[...], axis=1, keepdims=True)
            + jnp.sum(h2 * wo_n_ref[...], axis=1, keepdims=True))
    out_ref[...] = pred


def _dense_call(gu, nu, gi, ni, ga, na, gB, nB, W1, b1, W2, b2, Wo,
                interpret=False):
    n = gu.shape[0]
    grid = (n // _BLK,)
    row = lambda i: (i, 0)
    whole = lambda i: (0, 0)
    rowspec = lambda w: pl.BlockSpec((_BLK, w), row)
    out = pl.pallas_call(
        _dense_body,
        grid=grid,
        in_specs=[
            rowspec(DIM), rowspec(DIM), rowspec(DIM), rowspec(DIM),
            rowspec(RANK), rowspec(RANK),
            pl.BlockSpec((RANK, DIM), whole), pl.BlockSpec((RANK, DIM), whole),
            pl.BlockSpec((2 * DIM, DIM), whole), pl.BlockSpec((1, DIM), whole),
            pl.BlockSpec((DIM, RANK), whole), pl.BlockSpec((1, RANK), whole),
            pl.BlockSpec((1, DIM), whole), pl.BlockSpec((1, RANK), whole),
        ],
        out_specs=pl.BlockSpec((_BLK, 1), row),
        out_shape=jax.ShapeDtypeStruct((n, 1), jnp.float32),
        interpret=interpret,
    )(gu, nu, gi, ni, ga, na, gB, nB,
      W1.T, b1.reshape(1, DIM), W2.T, b2.reshape(1, RANK),
      Wo[:, :DIM], Wo[:, DIM:])
    return out[:, 0]


def kernel(users, items, gmf_user_table, ncf_user_table, gmf_item_table,
           ncf_item_table, gmf_A, ncf_A, gmf_B, ncf_B, W1, b1, W2, b2, Wo):
    users_r = users.astype(jnp.int32).reshape(_NW, _NCHUNK, _CHUNK)
    items_r = items.astype(jnp.int32).reshape(_NW, _NCHUNK, _CHUNK)
    gu, nu, gi, ni, ga, na = _sc_gather(
        users_r, items_r, gmf_user_table, ncf_user_table,
        gmf_item_table, ncf_item_table, gmf_A, ncf_A)
    return _dense_call(gu, nu, gi, ni, ga, na,
                       gmf_B, ncf_B, W1, b1, W2, b2, Wo)
